# Initial kernel scaffold; baseline (speedup 1.0000x reference)
import jax
import jax.numpy as jnp
from jax import lax
from jax.experimental import pallas as pl
from jax.experimental.pallas import tpu as pltpu
from jax.experimental.pallas import tpu_sc as plsc

SEQ = 200
DIM = 32
NUM_CORES = 2
NUM_SUBCORES = 16
NUM_WORKERS = NUM_CORES * NUM_SUBCORES
CHUNK = 128


def _pe_table():
    pos = jnp.arange(SEQ, dtype=jnp.float32)[:, None]
    j = jnp.arange(DIM, dtype=jnp.float32)[None, :]
    angle = pos / (10000.0 ** (j / float(DIM)))
    even = (jnp.arange(DIM)[None, :] % 2) == 0
    return jnp.where(even, jnp.sin(angle), jnp.cos(angle)).astype(jnp.float32)


def _sc_body(x_hbm, pe_hbm, table_hbm, out_hbm, idx_v, rows_v, pe_v, sem):
    wid = lax.axis_index("s") * NUM_CORES + lax.axis_index("c")
    rows_per_w = x_hbm.shape[0] // NUM_WORKERS
    n_chunks = rows_per_w // CHUNK
    base_w = wid * rows_per_w

    pltpu.sync_copy(pe_hbm, pe_v)

    def chunk_body(c, carry):
        base = base_w + c * CHUNK
        pltpu.sync_copy(x_hbm.at[pl.ds(base, CHUNK)], idx_v)
        pltpu.async_copy(table_hbm.at[idx_v], rows_v, sem).wait()
        p0 = lax.rem(c * CHUNK, SEQ)

        def row_body(r, carry2):
            s = lax.rem(p0 + r, SEQ)
            rows_v[r, 0:16] = rows_v[r, 0:16] + pe_v[s, 0:16]
            rows_v[r, 16:32] = rows_v[r, 16:32] + pe_v[s, 16:32]
            return carry2

        lax.fori_loop(0, CHUNK, row_body, 0, unroll=4)
        pltpu.sync_copy(rows_v, out_hbm.at[pl.ds(base, CHUNK)])
        return carry

    lax.fori_loop(0, n_chunks, chunk_body, 0)


@jax.jit
def kernel(x, table):
    b, seq = x.shape
    n = b * seq
    x_flat = x.reshape(n)
    pe = _pe_table()
    mesh = plsc.VectorSubcoreMesh(core_axis_name="c", subcore_axis_name="s")
    run = pl.kernel(
        _sc_body,
        out_type=jax.ShapeDtypeStruct((n, DIM), jnp.float32),
        mesh=mesh,
        scratch_types=[
            pltpu.VMEM((CHUNK,), jnp.int32),
            pltpu.VMEM((CHUNK, DIM), jnp.float32),
            pltpu.VMEM((SEQ, DIM), jnp.float32),
            pltpu.SemaphoreType.DMA,
        ],
    )
    out = run(x_flat, pe, table)
    return out.reshape(b, seq, DIM)


# speedup vs baseline: 1.0243x; 1.0243x over previous
import jax
import jax.numpy as jnp
from jax import lax
from jax.experimental import pallas as pl
from jax.experimental.pallas import tpu as pltpu
from jax.experimental.pallas import tpu_sc as plsc

SEQ = 200
DIM = 32
NUM_CORES = 2
NUM_SUBCORES = 16
NUM_WORKERS = NUM_CORES * NUM_SUBCORES
CHUNK = 128


def _pe_table():
    pos = jnp.arange(SEQ, dtype=jnp.float32)[:, None]
    j = jnp.arange(DIM, dtype=jnp.float32)[None, :]
    angle = pos / (10000.0 ** (j / float(DIM)))
    even = (jnp.arange(DIM)[None, :] % 2) == 0
    return jnp.where(even, jnp.sin(angle), jnp.cos(angle)).astype(jnp.float32)


def _sc_body(x_hbm, pe_hbm, table_hbm, out_hbm, idx_v, rows_v, pe_v, sem):
    wid = lax.axis_index("s") * NUM_CORES + lax.axis_index("c")
    rows_per_w = x_hbm.shape[0] // NUM_WORKERS
    n_chunks = rows_per_w // CHUNK
    base_w = wid * rows_per_w

    pltpu.sync_copy(pe_hbm, pe_v)

    def chunk_body(c, carry):
        base = base_w + c * CHUNK
        pltpu.sync_copy(x_hbm.at[pl.ds(base, CHUNK)], idx_v)
        pltpu.async_copy(table_hbm.at[idx_v], rows_v, sem).wait()
        p0 = lax.rem(c * CHUNK, SEQ)

        def row_body(r, carry2):
            s = lax.rem(p0 + r, SEQ)
            rows_v[r, 0:16] = rows_v[r, 0:16] + pe_v[s, 0:16]
            rows_v[r, 16:32] = rows_v[r, 16:32] + pe_v[s, 16:32]
            return carry2

        lax.fori_loop(0, CHUNK, row_body, 0, unroll=4)
        pltpu.sync_copy(rows_v, out_hbm.at[pl.ds(base, CHUNK)])
        return carry

    lax.fori_loop(0, n_chunks, chunk_body, 0)


@jax.jit
def kernel(x, table):
    b, seq = x.shape
    n = b * seq
    x_flat = x.reshape(n)
    pe = _pe_table()
    mesh = plsc.VectorSubcoreMesh(
        core_axis_name="c", subcore_axis_name="s",
        num_cores=NUM_CORES, num_subcores=NUM_SUBCORES)
    run = pl.kernel(
        _sc_body,
        out_type=jax.ShapeDtypeStruct((n, DIM), jnp.float32),
        mesh=mesh,
        scratch_types=[
            pltpu.VMEM((CHUNK,), jnp.int32),
            pltpu.VMEM((CHUNK, DIM), jnp.float32),
            pltpu.VMEM((SEQ, DIM), jnp.float32),
            pltpu.SemaphoreType.DMA,
        ],
        compiler_params=pltpu.CompilerParams(use_tc_tiling_on_sc=False),
    )
    out = run(x_flat, pe, table)
    return out.reshape(b, seq, DIM)


# speedup vs baseline: 1.2516x; 1.2218x over previous
import jax
import jax.numpy as jnp
from jax import lax
from jax.experimental import pallas as pl
from jax.experimental.pallas import tpu as pltpu
from jax.experimental.pallas import tpu_sc as plsc

SEQ = 200
DIM = 32
NUM_CORES = 2
NUM_SUBCORES = 16
NUM_WORKERS = NUM_CORES * NUM_SUBCORES
CHUNK = 128
NBUF = 4


def _pe_table():
    pos = jnp.arange(SEQ, dtype=jnp.float32)[:, None]
    j = jnp.arange(DIM, dtype=jnp.float32)[None, :]
    angle = pos / (10000.0 ** (j / float(DIM)))
    even = (jnp.arange(DIM)[None, :] % 2) == 0
    return jnp.where(even, jnp.sin(angle), jnp.cos(angle)).astype(jnp.float32)


def _sc_body(x_hbm, pe_hbm, table_hbm, out_hbm, idx_v, peb_v, rows_g, rows_o,
             *sems):
    sem_g = sems[:NBUF]
    sem_s = sems[NBUF:]
    wid = lax.axis_index("s") * NUM_CORES + lax.axis_index("c")
    n_rows = x_hbm.shape[0] * x_hbm.shape[1]
    rows_per_w = n_rows // NUM_WORKERS
    n_chunks = rows_per_w // CHUNK
    n_outer = n_chunks // NBUF
    base_w = wid * rows_per_w

    pltpu.sync_copy(x_hbm.at[pl.ds(wid * n_chunks, n_chunks)], idx_v)
    pltpu.sync_copy(pe_hbm, peb_v)

    def fire_gather(i, b):
        pltpu.async_copy(table_hbm.at[idx_v.at[i]], rows_g.at[b], sem_g[b])

    for b in range(NBUF):
        fire_gather(b, b)

    def outer_body(k, carry):
        for b in range(NBUF):
            i = k * NBUF + b
            row0 = base_w + i * CHUNK
            pltpu.make_async_copy(table_hbm.at[idx_v.at[i]], rows_g.at[b],
                                  sem_g[b]).wait()

            @pl.when(k > 0)
            def _wait_store():
                pltpu.make_async_copy(rows_o.at[b],
                                      out_hbm.at[pl.ds(row0, CHUNK)],
                                      sem_s[b]).wait()

            p0 = lax.rem(i * CHUNK, SEQ)

            def row_body(r, carry2):
                s = p0 + r
                rows_o[b, r, 0:16] = rows_g[b, r, 0:16] + peb_v[s, 0:16]
                rows_o[b, r, 16:32] = rows_g[b, r, 16:32] + peb_v[s, 16:32]
                return carry2

            lax.fori_loop(0, CHUNK, row_body, 0, unroll=8)

            pltpu.async_copy(rows_o.at[b], out_hbm.at[pl.ds(row0, CHUNK)],
                             sem_s[b])

            @pl.when(k < n_outer - 1)
            def _next_gather():
                fire_gather(i + NBUF, b)
        return carry

    lax.fori_loop(0, n_outer, outer_body, 0)

    for b in range(NBUF):
        i = (n_outer - 1) * NBUF + b
        row0 = base_w + i * CHUNK
        pltpu.make_async_copy(rows_o.at[b], out_hbm.at[pl.ds(row0, CHUNK)],
                              sem_s[b]).wait()


@jax.jit
def kernel(x, table):
    b, seq = x.shape
    n = b * seq
    x2d = x.reshape(n // CHUNK, CHUNK)
    pe = _pe_table()
    pe_wrap = jnp.concatenate([pe, pe[:CHUNK]], axis=0)
    n_chunks = n // NUM_WORKERS // CHUNK
    mesh = plsc.VectorSubcoreMesh(
        core_axis_name="c", subcore_axis_name="s",
        num_cores=NUM_CORES, num_subcores=NUM_SUBCORES)
    run = pl.kernel(
        _sc_body,
        out_type=jax.ShapeDtypeStruct((n, DIM), jnp.float32),
        mesh=mesh,
        scratch_types=[
            pltpu.VMEM((n_chunks, CHUNK), jnp.int32),
            pltpu.VMEM((SEQ + CHUNK, DIM), jnp.float32),
            pltpu.VMEM((NBUF, CHUNK, DIM), jnp.float32),
            pltpu.VMEM((NBUF, CHUNK, DIM), jnp.float32),
        ] + [pltpu.SemaphoreType.DMA] * (2 * NBUF),
        compiler_params=pltpu.CompilerParams(use_tc_tiling_on_sc=False),
    )
    out = run(x2d, pe_wrap, table)
    return out.reshape(b, seq, DIM)


# speedup vs baseline: 1.3074x; 1.0447x over previous
import jax
import jax.numpy as jnp
from jax import lax
from jax.experimental import pallas as pl
from jax.experimental.pallas import tpu as pltpu
from jax.experimental.pallas import tpu_sc as plsc

SEQ = 200
DIM = 32
NUM_CORES = 2
NUM_SUBCORES = 16
NUM_WORKERS = NUM_CORES * NUM_SUBCORES
G0 = 128
G1 = SEQ - G0
NBUF = 4


def _pe_table():
    pos = jnp.arange(SEQ, dtype=jnp.float32)[:, None]
    j = jnp.arange(DIM, dtype=jnp.float32)[None, :]
    angle = pos / (10000.0 ** (j / float(DIM)))
    even = (jnp.arange(DIM)[None, :] % 2) == 0
    return jnp.where(even, jnp.sin(angle), jnp.cos(angle)).astype(jnp.float32)


def _sc_body(x_hbm, pe_hbm, table_hbm, out_hbm, idx_v, pe_v, rows_g, rows_o,
             *sems):
    sem_g = sems[:NBUF]
    sem_s = sems[NBUF:]
    wid = lax.axis_index("s") * NUM_CORES + lax.axis_index("c")
    batch = x_hbm.shape[0]
    rows_per_w = batch // NUM_WORKERS
    n_outer = rows_per_w // NBUF
    base_w = wid * rows_per_w

    pltpu.sync_copy(x_hbm.at[pl.ds(base_w, rows_per_w)], idx_v)
    pltpu.sync_copy(pe_hbm, pe_v)

    def gather_copies(i, b):
        return (
            pltpu.make_async_copy(table_hbm.at[idx_v.at[i, pl.ds(0, G0)]],
                                  rows_g.at[b, pl.ds(0, G0)], sem_g[b]),
            pltpu.make_async_copy(table_hbm.at[idx_v.at[i, pl.ds(G0, G1)]],
                                  rows_g.at[b, pl.ds(G0, G1)], sem_g[b]),
        )

    for b in range(NBUF):
        for c in gather_copies(b, b):
            c.start()

    def outer_body(k, carry):
        for b in range(NBUF):
            i = k * NBUF + b
            for c in gather_copies(i, b):
                c.wait()

            @pl.when(k > 0)
            def _wait_store():
                pltpu.make_async_copy(rows_o.at[b], out_hbm.at[base_w + i],
                                      sem_s[b]).wait()

            def row_body(r, carry2):
                rows_o[b, r, 0:16] = rows_g[b, r, 0:16] + pe_v[r, 0:16]
                rows_o[b, r, 16:32] = rows_g[b, r, 16:32] + pe_v[r, 16:32]
                return carry2

            lax.fori_loop(0, SEQ, row_body, 0, unroll=8)

            pltpu.async_copy(rows_o.at[b], out_hbm.at[base_w + i], sem_s[b])

            @pl.when(k < n_outer - 1)
            def _next_gather():
                for c in gather_copies(i + NBUF, b):
                    c.start()
        return carry

    lax.fori_loop(0, n_outer, outer_body, 0)

    for b in range(NBUF):
        i = (n_outer - 1) * NBUF + b
        pltpu.make_async_copy(rows_o.at[b], out_hbm.at[base_w + i],
                              sem_s[b]).wait()


@jax.jit
def kernel(x, table):
    batch, seq = x.shape
    pe = _pe_table()
    rows_per_w = batch // NUM_WORKERS
    mesh = plsc.VectorSubcoreMesh(
        core_axis_name="c", subcore_axis_name="s",
        num_cores=NUM_CORES, num_subcores=NUM_SUBCORES)
    run = pl.kernel(
        _sc_body,
        out_type=jax.ShapeDtypeStruct((batch, seq, DIM), jnp.float32),
        mesh=mesh,
        scratch_types=[
            pltpu.VMEM((rows_per_w, SEQ), jnp.int32),
            pltpu.VMEM((SEQ, DIM), jnp.float32),
            pltpu.VMEM((NBUF, SEQ, DIM), jnp.float32),
            pltpu.VMEM((NBUF, SEQ, DIM), jnp.float32),
        ] + [pltpu.SemaphoreType.DMA] * (2 * NBUF),
        compiler_params=pltpu.CompilerParams(use_tc_tiling_on_sc=False),
    )
    return run(x, pe, table)


# speedup vs baseline: 1.3991x; 1.0701x over previous
import jax
import jax.numpy as jnp
from jax import lax
from jax.experimental import pallas as pl
from jax.experimental.pallas import tpu as pltpu
from jax.experimental.pallas import tpu_sc as plsc

SEQ = 200
DIM = 32
NUM_CORES = 2
NUM_SUBCORES = 16
NUM_WORKERS = NUM_CORES * NUM_SUBCORES
BCHUNK = 128
NBUF = 4


def _pe_table():
    pos = jnp.arange(SEQ, dtype=jnp.float32)[:, None]
    j = jnp.arange(DIM, dtype=jnp.float32)[None, :]
    angle = pos / (10000.0 ** (j / float(DIM)))
    even = (jnp.arange(DIM)[None, :] % 2) == 0
    return jnp.where(even, jnp.sin(angle), jnp.cos(angle)).astype(jnp.float32)


def _sc_body(x_hbm, pe_hbm, table_hbm, out_hbm, idx_v, pe_v, rows_g,
             *rest):
    rows_o = rest[:NBUF]
    sem_g = rest[NBUF:2 * NBUF]
    sem_s = rest[2 * NBUF:]
    wid = lax.axis_index("s") * NUM_CORES + lax.axis_index("c")
    n_outer = SEQ // NBUF

    pltpu.sync_copy(x_hbm.at[pl.ds(0, SEQ // 8), wid], idx_v)
    pltpu.sync_copy(pe_hbm, pe_v)

    def gather_copy(s, b):
        return pltpu.make_async_copy(
            table_hbm.at[idx_v.at[s // 8, s % 8]], rows_g.at[b], sem_g[b])

    def store_copies(s, b):
        return [
            pltpu.make_async_copy(rows_o[b].at[tr], out_hbm.at[s, tr, wid],
                                  sem_s[b])
            for tr in range(4)
        ]

    for b in range(NBUF):
        gather_copy(b, b).start()

    def outer_body(k, carry):
        for b in range(NBUF):
            s = k * NBUF + b
            gather_copy(s, b).wait()

            @pl.when(k > 0)
            def _wait_store():
                for c in store_copies(s, b):
                    c.wait()

            def col_body(c, c_vec):
                i16 = lax.iota(jnp.int32, 16)
                tr_lo = i16 // 8
                tr_hi = tr_lo + 2
                r_j = i16 % 8
                lo = rows_g[b, c, 0:16] + pe_v[s, 0:16]
                hi = rows_g[b, c, 16:32] + pe_v[s, 16:32]
                plsc.store_scatter(rows_o[b], [tr_lo, r_j, c_vec], lo)
                plsc.store_scatter(rows_o[b], [tr_hi, r_j, c_vec], hi)
                return c_vec + 1

            lax.fori_loop(0, BCHUNK, col_body,
                          lax.iota(jnp.int32, 16) * 0)

            for c in store_copies(s, b):
                c.start()

            @pl.when(k < n_outer - 1)
            def _next_gather():
                gather_copy(s + NBUF, b).start()
        return carry

    lax.fori_loop(0, n_outer, outer_body, 0)

    for b in range(NBUF):
        s = (n_outer - 1) * NBUF + b
        for c in store_copies(s, b):
            c.wait()


@jax.jit
def kernel(x, table):
    batch, seq = x.shape
    pe = _pe_table()
    x5d = (x.T.reshape(seq // 8, 8, batch // 128, 128)
           .transpose(0, 2, 1, 3))
    mesh = plsc.VectorSubcoreMesh(
        core_axis_name="c", subcore_axis_name="s",
        num_cores=NUM_CORES, num_subcores=NUM_SUBCORES)
    run = pl.kernel(
        _sc_body,
        out_type=jax.ShapeDtypeStruct((seq, DIM // 8, batch // 128, 8, 128),
                                      jnp.float32),
        mesh=mesh,
        scratch_types=[
            pltpu.VMEM((seq // 8, 8, BCHUNK), jnp.int32),
            pltpu.VMEM((SEQ, DIM), jnp.float32),
            pltpu.VMEM((NBUF, BCHUNK, DIM), jnp.float32),
        ] + [pltpu.VMEM((DIM // 8, 8, BCHUNK), jnp.float32)] * NBUF
          + [pltpu.SemaphoreType.DMA] * (2 * NBUF),
        compiler_params=pltpu.CompilerParams(use_tc_tiling_on_sc=False,
                                             needs_layout_passes=False),
    )
    out5d = run(x5d, pe, table)
    return out5d.transpose(2, 4, 0, 1, 3).reshape(batch, seq, DIM)
